# baseline (speedup 1.0000x reference)
import jax
import jax.numpy as jnp
from jax.experimental import pallas as pl
from jax.experimental.pallas import tpu as pltpu

_ROWS = 16384
_COLS = 256
_CH_ROWS = 512
_N_CH = _ROWS // _CH_ROWS


def _fire_body(t_hbm, q_hbm, t_out, q_out, *scratch):
    n = 2 * _N_CH
    bufs = scratch[:n]
    sems = scratch[n]

    jobs = []
    for src, dst in ((t_hbm, t_out), (q_hbm, q_out)):
        for c in range(_N_CH):
            sl = pl.ds(c * _CH_ROWS, _CH_ROWS)
            jobs.append((src.at[sl], dst.at[sl]))

    loads = []
    for j, (src, _) in enumerate(jobs):
        cp = pltpu.make_async_copy(src, bufs[j], sems.at[j])
        cp.start()
        loads.append(cp)
    stores = []
    for j, (_, dst) in enumerate(jobs):
        loads[j].wait()
        cp = pltpu.make_async_copy(bufs[j], dst, sems.at[n + j])
        cp.start()
        stores.append(cp)
    for cp in stores:
        cp.wait()


def kernel(emb_targets, emb_queries):
    any_spec = pl.BlockSpec(memory_space=pl.MemorySpace.ANY)
    n = 2 * _N_CH
    out_t, out_q = pl.pallas_call(
        _fire_body,
        in_specs=[any_spec, any_spec],
        out_specs=[any_spec, any_spec],
        out_shape=[
            jax.ShapeDtypeStruct((_ROWS, _COLS), jnp.float32),
            jax.ShapeDtypeStruct((_ROWS, _COLS), jnp.float32),
        ],
        scratch_shapes=(
            [pltpu.VMEM((_CH_ROWS, _COLS), jnp.float32) for _ in range(n)]
            + [pltpu.SemaphoreType.DMA((2 * n,))]
        ),
        compiler_params=pltpu.CompilerParams(vmem_limit_bytes=100 * 1024 * 1024),
    )(emb_targets, emb_queries)
    return (out_t, out_q)


# speedup vs baseline: 1.0337x; 1.0337x over previous
import jax
import jax.numpy as jnp
from jax.experimental import pallas as pl
from jax.experimental.pallas import tpu as pltpu

_ROWS = 16384
_COLS = 256
_CH_ROWS = 2048
_N_CH = _ROWS // _CH_ROWS


def _fire_body(t_hbm, q_hbm, t_out, q_out, *scratch):
    n = 2 * _N_CH
    bufs = scratch[:n]
    sems = scratch[n]

    jobs = []
    for src, dst in ((t_hbm, t_out), (q_hbm, q_out)):
        for c in range(_N_CH):
            sl = pl.ds(c * _CH_ROWS, _CH_ROWS)
            jobs.append((src.at[sl], dst.at[sl]))

    loads = []
    for j, (src, _) in enumerate(jobs):
        cp = pltpu.make_async_copy(src, bufs[j], sems.at[j])
        cp.start()
        loads.append(cp)
    stores = []
    for j, (_, dst) in enumerate(jobs):
        loads[j].wait()
        cp = pltpu.make_async_copy(bufs[j], dst, sems.at[n + j])
        cp.start()
        stores.append(cp)
    for cp in stores:
        cp.wait()


def kernel(emb_targets, emb_queries):
    any_spec = pl.BlockSpec(memory_space=pl.MemorySpace.ANY)
    n = 2 * _N_CH
    out_t, out_q = pl.pallas_call(
        _fire_body,
        in_specs=[any_spec, any_spec],
        out_specs=[any_spec, any_spec],
        out_shape=[
            jax.ShapeDtypeStruct((_ROWS, _COLS), jnp.float32),
            jax.ShapeDtypeStruct((_ROWS, _COLS), jnp.float32),
        ],
        scratch_shapes=(
            [pltpu.VMEM((_CH_ROWS, _COLS), jnp.float32) for _ in range(n)]
            + [pltpu.SemaphoreType.DMA((2 * n,))]
        ),
        compiler_params=pltpu.CompilerParams(vmem_limit_bytes=100 * 1024 * 1024),
    )(emb_targets, emb_queries)
    return (out_t, out_q)


# speedup vs baseline: 1.0759x; 1.0408x over previous
import jax
import jax.numpy as jnp
from jax.experimental import pallas as pl
from jax.experimental.pallas import tpu as pltpu

_ROWS = 16384
_COLS = 256
_BLOCK_ROWS = 8184


def _copy_body(t_ref, q_ref, t_out, q_out):
    t_out[...] = t_ref[...]
    q_out[...] = q_ref[...]


def kernel(emb_targets, emb_queries):
    grid = (-(-_ROWS // _BLOCK_ROWS),)
    spec = pl.BlockSpec((_BLOCK_ROWS, _COLS), lambda i: (i, 0))
    out_t, out_q = pl.pallas_call(
        _copy_body,
        grid=grid,
        in_specs=[spec, spec],
        out_specs=[spec, spec],
        out_shape=[
            jax.ShapeDtypeStruct((_ROWS, _COLS), jnp.float32),
            jax.ShapeDtypeStruct((_ROWS, _COLS), jnp.float32),
        ],
        compiler_params=pltpu.CompilerParams(vmem_limit_bytes=100 * 1024 * 1024),
    )(emb_targets, emb_queries)
    return (out_t, out_q)


# speedup vs baseline: 1.1298x; 1.0501x over previous
import jax
import jax.numpy as jnp
from jax.experimental import pallas as pl
from jax.experimental.pallas import tpu as pltpu

_ROWS = 16384
_COLS = 256
_BLOCK_ROWS = 7680


def _copy_body(t_ref, q_ref, t_out, q_out):
    t_out[...] = t_ref[...]
    q_out[...] = q_ref[...]


def kernel(emb_targets, emb_queries):
    grid = (-(-_ROWS // _BLOCK_ROWS),)
    spec = pl.BlockSpec((_BLOCK_ROWS, _COLS), lambda i: (i, 0))
    out_t, out_q = pl.pallas_call(
        _copy_body,
        grid=grid,
        in_specs=[spec, spec],
        out_specs=[spec, spec],
        out_shape=[
            jax.ShapeDtypeStruct((_ROWS, _COLS), jnp.float32),
            jax.ShapeDtypeStruct((_ROWS, _COLS), jnp.float32),
        ],
        compiler_params=pltpu.CompilerParams(vmem_limit_bytes=100 * 1024 * 1024),
    )(emb_targets, emb_queries)
    return (out_t, out_q)
